# baseline (speedup 1.0000x reference)
import functools

import jax
import jax.numpy as jnp
from jax import lax
from jax.experimental import pallas as pl
from jax.experimental.pallas import tpu as pltpu
from jax.experimental.pallas import tpu_sc as plsc

NUM_RELATIONS = 1000
D = 64
DP = D // 2
NTB = 4
TBR = DP // NTB
B = 16384
NC = 2
NS = 16
L = 16
NW = NC * NS
BPW = B // NW

_mesh = plsc.VectorSubcoreMesh(core_axis_name="c", subcore_axis_name="s")


@functools.partial(
    pl.kernel,
    mesh=_mesh,
    out_type=jax.ShapeDtypeStruct((B,), jnp.float32),
    compiler_params=pltpu.CompilerParams(needs_layout_passes=False),
    scratch_types=[
        pltpu.VMEM((BPW,), jnp.int32),
        pltpu.VMEM((DP, NUM_RELATIONS), jnp.int32),
        pltpu.VMEM((D, BPW), jnp.float32),
        pltpu.VMEM((D, BPW), jnp.float32),
        pltpu.VMEM((BPW,), jnp.float32),
        pltpu.SemaphoreType.DMA,
    ] + [pltpu.SemaphoreType.DMA] * NTB,
)
def _distmult_sc(head_hbm, tail_hbm, idx_hbm, table_hbm, out_hbm,
                 idx_v, table_v, head_v, tail_v, out_v, sem0, *tsems):
    wid = lax.axis_index("s") * NC + lax.axis_index("c")
    base = wid * BPW

    idx_cp = pltpu.async_copy(idx_hbm.at[pl.ds(base, BPW)], idx_v, sem0)
    blocks = []
    for tb in range(NTB):
        dp0, d0 = tb * TBR, tb * TBR * 2
        blocks.append((
            pltpu.async_copy(table_hbm.at[pl.ds(dp0, TBR)],
                             table_v.at[pl.ds(dp0, TBR)], tsems[tb]),
            pltpu.async_copy(head_hbm.at[pl.ds(d0, 2 * TBR),
                                         pl.ds(base, BPW)],
                             head_v.at[pl.ds(d0, 2 * TBR)], tsems[tb]),
            pltpu.async_copy(tail_hbm.at[pl.ds(d0, 2 * TBR),
                                         pl.ds(base, BPW)],
                             tail_v.at[pl.ds(d0, 2 * TBR)], tsems[tb]),
        ))

    himask = jnp.full((L,), jnp.int32(-65536))

    with jax.named_scope("idxwait"):
        idx_cp.wait()
    for tb in range(NTB):
        with jax.named_scope(f"wait{tb}"):
            for cp in blocks[tb]:
                cp.wait()
        scope = jax.named_scope(f"comp{tb}")
        scope.__enter__()

        @plsc.parallel_loop(0, BPW // L, unroll=4)
        def _j_body(j, tb=tb):
            b0 = j * L
            idxv = idx_v[pl.ds(b0, L)]
            accs = [jnp.zeros((L,), jnp.float32) for _ in range(4)]
            for dp in range(tb * TBR, (tb + 1) * TBR):
                pv = plsc.load_gather(
                    table_v, [jnp.full((L,), dp, jnp.int32), idxv])
                r_lo = plsc.bitcast(lax.shift_left(pv, 16), jnp.float32)
                r_hi = plsc.bitcast(lax.bitwise_and(pv, himask), jnp.float32)
                d0 = 2 * dp
                h0 = head_v[d0, pl.ds(b0, L)]
                t0 = tail_v[d0, pl.ds(b0, L)]
                h1 = head_v[d0 + 1, pl.ds(b0, L)]
                t1 = tail_v[d0 + 1, pl.ds(b0, L)]
                accs[dp % 4] = accs[dp % 4] + (h0 * r_lo * t0 + h1 * r_hi * t1)
            s = (accs[0] + accs[1]) + (accs[2] + accs[3])
            if tb > 0:
                s = s + out_v[pl.ds(b0, L)]
            out_v[pl.ds(b0, L)] = s
        scope.__exit__(None, None, None)

    pltpu.sync_copy(out_v, out_hbm.at[pl.ds(base, BPW)])


def kernel(head_emb, tail_emb, rel_idx, relation_embeddings):
    idx = rel_idx.astype(jnp.int32)
    ru = lax.bitcast_convert_type(relation_embeddings, jnp.uint32)
    rb = (ru + jnp.uint32(0x8000)) >> jnp.uint32(16)
    packed_u = rb[:, 0::2] | (rb[:, 1::2] << jnp.uint32(16))
    packed = lax.bitcast_convert_type(packed_u, jnp.int32).T
    return _distmult_sc(head_emb.T, tail_emb.T, idx, packed)


# speedup vs baseline: 1.2583x; 1.2583x over previous
import functools

import jax
import jax.numpy as jnp
from jax import lax
from jax.experimental import pallas as pl
from jax.experimental.pallas import tpu as pltpu
from jax.experimental.pallas import tpu_sc as plsc

NUM_RELATIONS = 1000
D = 64
DP = D // 2
NTB = 4
TBR = DP // NTB
B = 16384
NC = 2
NS = 16
L = 16
NW = NC * NS
BPW = B // NW

_mesh = plsc.VectorSubcoreMesh(core_axis_name="c", subcore_axis_name="s")


@functools.partial(
    pl.kernel,
    mesh=_mesh,
    out_type=jax.ShapeDtypeStruct((B,), jnp.float32),
    compiler_params=pltpu.CompilerParams(needs_layout_passes=False),
    scratch_types=[
        pltpu.VMEM((BPW,), jnp.int32),
        pltpu.VMEM((DP, NUM_RELATIONS), jnp.int32),
        pltpu.VMEM((D, BPW), jnp.float32),
        pltpu.VMEM((D, BPW), jnp.float32),
        pltpu.VMEM((BPW,), jnp.float32),
        pltpu.SemaphoreType.DMA,
    ] + [pltpu.SemaphoreType.DMA] * NTB,
)
def _distmult_sc(head_hbm, tail_hbm, idx_hbm, table_hbm, out_hbm,
                 idx_v, table_v, head_v, tail_v, out_v, sem0, *tsems):
    wid = lax.axis_index("s") * NC + lax.axis_index("c")
    base = wid * BPW

    idx_cp = pltpu.async_copy(idx_hbm.at[pl.ds(base, BPW)], idx_v, sem0)
    blocks = []
    for tb in range(NTB):
        dp0, d0 = tb * TBR, tb * TBR * 2
        blocks.append((
            pltpu.async_copy(table_hbm.at[pl.ds(dp0, TBR)],
                             table_v.at[pl.ds(dp0, TBR)], tsems[tb]),
            pltpu.async_copy(head_hbm.at[pl.ds(d0, 2 * TBR),
                                         pl.ds(base, BPW)],
                             head_v.at[pl.ds(d0, 2 * TBR)], tsems[tb]),
            pltpu.async_copy(tail_hbm.at[pl.ds(d0, 2 * TBR),
                                         pl.ds(base, BPW)],
                             tail_v.at[pl.ds(d0, 2 * TBR)], tsems[tb]),
        ))

    himask = jnp.full((L,), jnp.int32(-65536))

    with jax.named_scope("idxwait"):
        idx_cp.wait()
    for tb in range(NTB):
        with jax.named_scope(f"wait{tb}"):
            for cp in blocks[tb]:
                cp.wait()
        scope = jax.named_scope(f"comp{tb}")
        scope.__enter__()

        @plsc.parallel_loop(0, BPW // L, unroll=2)
        def _j_body(j, tb=tb):
            b0 = j * L
            idxv = idx_v[pl.ds(b0, L)]
            accs = [jnp.zeros((L,), jnp.float32) for _ in range(4)]
            for dp in range(tb * TBR, (tb + 1) * TBR):
                pv = plsc.load_gather(
                    table_v, [jnp.full((L,), dp, jnp.int32), idxv])
                r_lo = plsc.bitcast(lax.shift_left(pv, 16), jnp.float32)
                r_hi = plsc.bitcast(lax.bitwise_and(pv, himask), jnp.float32)
                d0 = 2 * dp
                h0 = head_v[d0, pl.ds(b0, L)]
                t0 = tail_v[d0, pl.ds(b0, L)]
                h1 = head_v[d0 + 1, pl.ds(b0, L)]
                t1 = tail_v[d0 + 1, pl.ds(b0, L)]
                accs[dp % 4] = accs[dp % 4] + (h0 * r_lo * t0 + h1 * r_hi * t1)
            s = (accs[0] + accs[1]) + (accs[2] + accs[3])
            if tb > 0:
                s = s + out_v[pl.ds(b0, L)]
            out_v[pl.ds(b0, L)] = s
        scope.__exit__(None, None, None)

    pltpu.sync_copy(out_v, out_hbm.at[pl.ds(base, BPW)])


def kernel(head_emb, tail_emb, rel_idx, relation_embeddings):
    idx = rel_idx.astype(jnp.int32)
    ru = lax.bitcast_convert_type(relation_embeddings, jnp.uint32)
    rb = (ru + jnp.uint32(0x8000)) >> jnp.uint32(16)
    packed_u = rb[:, 0::2] | (rb[:, 1::2] << jnp.uint32(16))
    packed = lax.bitcast_convert_type(packed_u, jnp.int32).T
    return _distmult_sc(head_emb.T, tail_emb.T, idx, packed)
